# baseline (speedup 1.0000x reference)
import functools

import jax
import jax.numpy as jnp
from jax import lax
from jax.experimental import pallas as pl
from jax.experimental.pallas import tpu as pltpu
from jax.experimental.pallas import tpu_sc as plsc

_BATCH = 4096
_HIST = 200
_DIM = 32
_V1 = 1000001
_VP = 1000064
_CHUNK = 4
_ROWS = _CHUNK * _HIST
_K = 1024
_C0_MAX = _VP - _K


def _make_transpose(nc, ns):
    mesh = plsc.VectorSubcoreMesh(core_axis_name="c", subcore_axis_name="s")
    nw = nc * ns
    n_slots = -(-(_V1 // _K + 1) // nw)

    @functools.partial(
        pl.kernel,
        mesh=mesh,
        compiler_params=pltpu.CompilerParams(
            disable_bounds_checks=True, needs_layout_passes=False),
        out_type=jax.ShapeDtypeStruct((_VP * _DIM,), jnp.float32),
        scratch_types=[
            pltpu.VMEM((_DIM, _K), jnp.float32),
            pltpu.VMEM((_K * _DIM,), jnp.float32),
        ],
    )
    def k(tt_hbm, flat_hbm, buf, out_v):
        wid = lax.axis_index("s") * nc + lax.axis_index("c")
        lane32 = lax.iota(jnp.int32, 16) * _DIM

        def chunk(ci, carry):
            g = wid * n_slots + ci
            c0 = jnp.minimum(g * _K, _C0_MAX)
            pltpu.sync_copy(tt_hbm.at[:, pl.ds(c0, _K)], buf)

            def gloop(gg, carry2):
                gbase = gg * (16 * _DIM) + lane32
                for d in range(_DIM):
                    v = buf[d, pl.ds(gg * 16, 16)]
                    plsc.store_scatter(out_v, [gbase + d], v)
                return carry2

            lax.fori_loop(0, _K // 16, gloop, 0)
            pltpu.sync_copy(out_v, flat_hbm.at[pl.ds(c0 * _DIM, _K * _DIM)])
            return carry

        lax.fori_loop(0, n_slots, chunk, 0)

    return k


def _make_gather(nc, ns, bpw):
    mesh = plsc.VectorSubcoreMesh(core_axis_name="c", subcore_axis_name="s")
    n_chunks = bpw // _CHUNK

    @functools.partial(
        pl.kernel,
        mesh=mesh,
        compiler_params=pltpu.CompilerParams(use_tc_tiling_on_sc=False),
        out_type=jax.ShapeDtypeStruct((_BATCH, _DIM), jnp.float32),
        scratch_types=[
            pltpu.VMEM((bpw * _HIST,), jnp.int32),
            pltpu.VMEM((bpw, 16), jnp.float32),
            pltpu.VMEM((_ROWS, _DIM), jnp.float32),
            pltpu.VMEM((_ROWS, _DIM), jnp.float32),
            pltpu.VMEM((bpw, _DIM), jnp.float32),
            pltpu.SemaphoreType.DMA,
            pltpu.SemaphoreType.DMA,
        ],
    )
    def k(idx_hbm, lens_hbm, table_hbm, out_hbm,
          idx_v, lens_v, buf0, buf1, out_v, sem0, sem1):
        wid = lax.axis_index("s") * nc + lax.axis_index("c")
        base = wid * bpw
        pltpu.sync_copy(idx_hbm.at[wid], idx_v)
        pltpu.sync_copy(lens_hbm.at[pl.ds(base, bpw)], lens_v)

        def gather(cc, buf, sem):
            return pltpu.async_copy(
                table_hbm.at[idx_v.at[pl.ds(cc * _ROWS, _ROWS)]], buf, sem)

        gather(0, buf0, sem0)

        def super_body(g, carry):
            for b in range(2):
                cc = 2 * g + b
                bufc, semc = (buf0, sem0) if b == 0 else (buf1, sem1)
                bufn, semn = (buf1, sem1) if b == 0 else (buf0, sem0)

                @pl.when(cc + 1 < n_chunks)
                def _():
                    gather(cc + 1, bufn, semn)

                pltpu.make_async_copy(
                    table_hbm.at[idx_v.at[pl.ds(cc * _ROWS, _ROWS)]],
                    bufc, semc).wait()

                for jj in range(_CHUNK):
                    rbase = jj * _HIST

                    def red(l, acc):
                        accs = list(acc)
                        r0 = rbase + l * 8
                        for t in range(8):
                            p = t % 4
                            accs[2 * p] = accs[2 * p] + bufc[r0 + t, pl.ds(0, 16)]
                            accs[2 * p + 1] = (
                                accs[2 * p + 1] + bufc[r0 + t, pl.ds(16, 16)])
                        return tuple(accs)

                    zero = jnp.zeros((16,), jnp.float32)
                    accs = lax.fori_loop(0, _HIST // 8, red, (zero,) * 8)
                    a0 = (accs[0] + accs[2]) + (accs[4] + accs[6])
                    a1 = (accs[1] + accs[3]) + (accs[5] + accs[7])
                    j = cc * _CHUNK + jj
                    lenv = lens_v[j, pl.ds(0, 16)]
                    out_v[j, pl.ds(0, 16)] = a0 / lenv
                    out_v[j, pl.ds(16, 16)] = a1 / lenv
            return carry

        lax.fori_loop(0, n_chunks // 2, super_body, 0)
        pltpu.sync_copy(out_v, out_hbm.at[pl.ds(base, bpw)])

    return k


def kernel(inputs, input_lens, table):
    info = plsc.get_sparse_core_info()
    nc, ns = info.num_cores, info.num_subcores
    nw = nc * ns
    bpw = _BATCH // nw
    idx = inputs.reshape(nw, bpw * _HIST)
    lens = jnp.broadcast_to(input_lens.reshape(_BATCH, 1), (_BATCH, 16))
    tt = jnp.transpose(table)
    flat = _make_transpose(nc, ns)(tt)
    tbl = flat.reshape(_VP, _DIM)
    return _make_gather(nc, ns, bpw)(idx, lens, tbl)


# speedup vs baseline: 4.2710x; 4.2710x over previous
import functools

import jax
import jax.numpy as jnp
from jax import lax
from jax.experimental import pallas as pl
from jax.experimental.pallas import tpu as pltpu
from jax.experimental.pallas import tpu_sc as plsc

_BATCH = 4096
_HIST = 200
_DIM = 32
_V1 = 1000001
_VP = 1000064
_CHUNK = 4
_ROWS = _CHUNK * _HIST
_K = 1024
_C0_MAX = _VP - _K


def _make_transpose(nc, ns):
    mesh = plsc.VectorSubcoreMesh(core_axis_name="c", subcore_axis_name="s")
    nw = nc * ns
    n_slots = -(-(_V1 // _K + 1) // nw)

    @functools.partial(
        pl.kernel,
        mesh=mesh,
        compiler_params=pltpu.CompilerParams(
            disable_bounds_checks=True, needs_layout_passes=False),
        out_type=jax.ShapeDtypeStruct((_VP * _DIM,), jnp.float32),
        scratch_types=[
            pltpu.VMEM((_DIM, _K), jnp.float32),
            pltpu.VMEM((_K * _DIM,), jnp.float32),
        ],
    )
    def k(tt_hbm, flat_hbm, buf, out_v):
        wid = lax.axis_index("s") * nc + lax.axis_index("c")
        lane32 = lax.iota(jnp.int32, 16) * _DIM

        def chunk(ci, carry):
            g = wid * n_slots + ci
            c0 = jnp.minimum(g * _K, _C0_MAX)
            pltpu.sync_copy(tt_hbm.at[:, pl.ds(c0, _K)], buf)

            @functools.partial(
                plsc.parallel_loop, 0, (_K // 16) * _DIM, unroll=8)
            def _(i):
                d = i & (_DIM - 1)
                gg = i >> 5
                v = buf[d, pl.ds(gg * 16, 16)]
                plsc.store_scatter(out_v, [gg * (16 * _DIM) + lane32 + d], v)

            pltpu.sync_copy(out_v, flat_hbm.at[pl.ds(c0 * _DIM, _K * _DIM)])
            return carry

        lax.fori_loop(0, n_slots, chunk, 0)

    return k


def _make_gather(nc, ns, bpw):
    mesh = plsc.VectorSubcoreMesh(core_axis_name="c", subcore_axis_name="s")
    n_chunks = bpw // _CHUNK

    @functools.partial(
        pl.kernel,
        mesh=mesh,
        compiler_params=pltpu.CompilerParams(use_tc_tiling_on_sc=False),
        out_type=jax.ShapeDtypeStruct((_BATCH, _DIM), jnp.float32),
        scratch_types=[
            pltpu.VMEM((bpw * _HIST,), jnp.int32),
            pltpu.VMEM((bpw, 16), jnp.float32),
            pltpu.VMEM((_ROWS, _DIM), jnp.float32),
            pltpu.VMEM((_ROWS, _DIM), jnp.float32),
            pltpu.VMEM((bpw, _DIM), jnp.float32),
            pltpu.SemaphoreType.DMA,
            pltpu.SemaphoreType.DMA,
        ],
    )
    def k(idx_hbm, lens_hbm, table_hbm, out_hbm,
          idx_v, lens_v, buf0, buf1, out_v, sem0, sem1):
        wid = lax.axis_index("s") * nc + lax.axis_index("c")
        base = wid * bpw
        pltpu.sync_copy(idx_hbm.at[wid], idx_v)
        pltpu.sync_copy(lens_hbm.at[pl.ds(base, bpw)], lens_v)

        def gather(cc, buf, sem):
            return pltpu.async_copy(
                table_hbm.at[idx_v.at[pl.ds(cc * _ROWS, _ROWS)]], buf, sem)

        gather(0, buf0, sem0)

        def super_body(g, carry):
            for b in range(2):
                cc = 2 * g + b
                bufc, semc = (buf0, sem0) if b == 0 else (buf1, sem1)
                bufn, semn = (buf1, sem1) if b == 0 else (buf0, sem0)

                @pl.when(cc + 1 < n_chunks)
                def _():
                    gather(cc + 1, bufn, semn)

                pltpu.make_async_copy(
                    table_hbm.at[idx_v.at[pl.ds(cc * _ROWS, _ROWS)]],
                    bufc, semc).wait()

                for jj in range(_CHUNK):
                    rbase = jj * _HIST

                    def red(l, acc):
                        accs = list(acc)
                        r0 = rbase + l * 8
                        for t in range(8):
                            p = t % 4
                            accs[2 * p] = accs[2 * p] + bufc[r0 + t, pl.ds(0, 16)]
                            accs[2 * p + 1] = (
                                accs[2 * p + 1] + bufc[r0 + t, pl.ds(16, 16)])
                        return tuple(accs)

                    zero = jnp.zeros((16,), jnp.float32)
                    accs = lax.fori_loop(0, _HIST // 8, red, (zero,) * 8)
                    a0 = (accs[0] + accs[2]) + (accs[4] + accs[6])
                    a1 = (accs[1] + accs[3]) + (accs[5] + accs[7])
                    j = cc * _CHUNK + jj
                    lenv = lens_v[j, pl.ds(0, 16)]
                    out_v[j, pl.ds(0, 16)] = a0 / lenv
                    out_v[j, pl.ds(16, 16)] = a1 / lenv
            return carry

        lax.fori_loop(0, n_chunks // 2, super_body, 0)
        pltpu.sync_copy(out_v, out_hbm.at[pl.ds(base, bpw)])

    return k


def kernel(inputs, input_lens, table):
    info = plsc.get_sparse_core_info()
    nc, ns = info.num_cores, info.num_subcores
    nw = nc * ns
    bpw = _BATCH // nw
    idx = inputs.reshape(nw, bpw * _HIST)
    lens = jnp.broadcast_to(input_lens.reshape(_BATCH, 1), (_BATCH, 16))
    tt = jnp.transpose(table)
    flat = _make_transpose(nc, ns)(tt)
    tbl = flat.reshape(_VP, _DIM)
    return _make_gather(nc, ns, bpw)(idx, lens, tbl)


# speedup vs baseline: 4.7085x; 1.1024x over previous
import functools

import jax
import jax.numpy as jnp
from jax import lax
from jax.experimental import pallas as pl
from jax.experimental.pallas import tpu as pltpu
from jax.experimental.pallas import tpu_sc as plsc

_BATCH = 4096
_HIST = 200
_DIM = 32
_V1 = 1000001
_VP = 1000064
_CHUNK = 4
_ROWS = _CHUNK * _HIST
_K = 896
_C0_MAX = _VP - _K


def _make_transpose(nc, ns):
    mesh = plsc.VectorSubcoreMesh(core_axis_name="c", subcore_axis_name="s")
    nw = nc * ns
    n_chunks = -(-_VP // _K)
    n_slots = -(-n_chunks // nw)
    n_slots += n_slots % 2
    n_pairs = n_slots // 2

    @functools.partial(
        pl.kernel,
        mesh=mesh,
        compiler_params=pltpu.CompilerParams(
            disable_bounds_checks=True, needs_layout_passes=False),
        out_type=jax.ShapeDtypeStruct((_VP * _DIM,), jnp.float32),
        scratch_types=[
            pltpu.VMEM((_DIM, _K), jnp.float32),
            pltpu.VMEM((_DIM, _K), jnp.float32),
            pltpu.VMEM((_K * _DIM,), jnp.float32),
            pltpu.VMEM((_K * _DIM,), jnp.float32),
            pltpu.SemaphoreType.DMA,
            pltpu.SemaphoreType.DMA,
            pltpu.SemaphoreType.DMA,
            pltpu.SemaphoreType.DMA,
        ],
    )
    def k(tt_hbm, flat_hbm, buf0, buf1, out0, out1, si0, si1, so0, so1):
        wid = lax.axis_index("s") * nc + lax.axis_index("c")
        lane32 = lax.iota(jnp.int32, 16) * _DIM
        bufs = (buf0, buf1)
        outs = (out0, out1)
        sis = (si0, si1)
        sos = (so0, so1)

        def c_of(ci):
            return jnp.minimum((wid * n_slots + ci) * _K, _C0_MAX)

        def start_in(ci, b):
            pltpu.async_copy(tt_hbm.at[:, pl.ds(c_of(ci), _K)], bufs[b], sis[b])

        def wait_in(ci, b):
            pltpu.make_async_copy(
                tt_hbm.at[:, pl.ds(c_of(ci), _K)], bufs[b], sis[b]).wait()

        def start_out(ci, b):
            pltpu.async_copy(
                outs[b], flat_hbm.at[pl.ds(c_of(ci) * _DIM, _K * _DIM)], sos[b])

        def wait_out(ci, b):
            pltpu.make_async_copy(
                outs[b], flat_hbm.at[pl.ds(c_of(ci) * _DIM, _K * _DIM)],
                sos[b]).wait()

        def pair(g, carry):
            for b in range(2):
                ci = 2 * g + b

                start_in(ci, b)
                wait_in(ci, b)

                @pl.when(ci >= 2)
                def _():
                    wait_out(ci - 2, b)

                @functools.partial(
                    plsc.parallel_loop, 0, (_K // 16) * _DIM, unroll=8)
                def _(i):
                    d = i & (_DIM - 1)
                    gg = i >> 5
                    v = bufs[b][d, pl.ds(gg * 16, 16)]
                    plsc.store_scatter(
                        outs[b], [gg * (16 * _DIM) + lane32 + d], v)

                start_out(ci, b)
            return carry

        lax.fori_loop(0, n_pairs, pair, 0)
        wait_out(n_slots - 2, 0)
        wait_out(n_slots - 1, 1)

    return k


def _make_gather(nc, ns, bpw):
    mesh = plsc.VectorSubcoreMesh(core_axis_name="c", subcore_axis_name="s")
    n_chunks = bpw // _CHUNK

    @functools.partial(
        pl.kernel,
        mesh=mesh,
        compiler_params=pltpu.CompilerParams(use_tc_tiling_on_sc=False),
        out_type=jax.ShapeDtypeStruct((_BATCH, _DIM), jnp.float32),
        scratch_types=[
            pltpu.VMEM((bpw * _HIST,), jnp.int32),
            pltpu.VMEM((bpw, 16), jnp.float32),
            pltpu.VMEM((_ROWS, _DIM), jnp.float32),
            pltpu.VMEM((_ROWS, _DIM), jnp.float32),
            pltpu.VMEM((bpw, _DIM), jnp.float32),
            pltpu.SemaphoreType.DMA,
            pltpu.SemaphoreType.DMA,
        ],
    )
    def k(idx_hbm, lens_hbm, table_hbm, out_hbm,
          idx_v, lens_v, buf0, buf1, out_v, sem0, sem1):
        wid = lax.axis_index("s") * nc + lax.axis_index("c")
        base = wid * bpw
        pltpu.sync_copy(idx_hbm.at[wid], idx_v)
        pltpu.sync_copy(lens_hbm.at[pl.ds(base, bpw)], lens_v)

        def gather(cc, buf, sem):
            return pltpu.async_copy(
                table_hbm.at[idx_v.at[pl.ds(cc * _ROWS, _ROWS)]], buf, sem)

        gather(0, buf0, sem0)

        def super_body(g, carry):
            for b in range(2):
                cc = 2 * g + b
                bufc, semc = (buf0, sem0) if b == 0 else (buf1, sem1)
                bufn, semn = (buf1, sem1) if b == 0 else (buf0, sem0)

                @pl.when(cc + 1 < n_chunks)
                def _():
                    gather(cc + 1, bufn, semn)

                pltpu.make_async_copy(
                    table_hbm.at[idx_v.at[pl.ds(cc * _ROWS, _ROWS)]],
                    bufc, semc).wait()

                for jj in range(_CHUNK):
                    rbase = jj * _HIST

                    def red(l, acc):
                        accs = list(acc)
                        r0 = rbase + l * 8
                        for t in range(8):
                            p = t % 4
                            accs[2 * p] = accs[2 * p] + bufc[r0 + t, pl.ds(0, 16)]
                            accs[2 * p + 1] = (
                                accs[2 * p + 1] + bufc[r0 + t, pl.ds(16, 16)])
                        return tuple(accs)

                    zero = jnp.zeros((16,), jnp.float32)
                    accs = lax.fori_loop(0, _HIST // 8, red, (zero,) * 8)
                    a0 = (accs[0] + accs[2]) + (accs[4] + accs[6])
                    a1 = (accs[1] + accs[3]) + (accs[5] + accs[7])
                    j = cc * _CHUNK + jj
                    lenv = lens_v[j, pl.ds(0, 16)]
                    out_v[j, pl.ds(0, 16)] = a0 / lenv
                    out_v[j, pl.ds(16, 16)] = a1 / lenv
            return carry

        lax.fori_loop(0, n_chunks // 2, super_body, 0)
        pltpu.sync_copy(out_v, out_hbm.at[pl.ds(base, bpw)])

    return k


def kernel(inputs, input_lens, table):
    info = plsc.get_sparse_core_info()
    nc, ns = info.num_cores, info.num_subcores
    nw = nc * ns
    bpw = _BATCH // nw
    idx = inputs.reshape(nw, bpw * _HIST)
    lens = jnp.broadcast_to(input_lens.reshape(_BATCH, 1), (_BATCH, 16))
    tt = jnp.transpose(table)
    flat = _make_transpose(nc, ns)(tt)
    tbl = flat.reshape(_VP, _DIM)
    return _make_gather(nc, ns, bpw)(idx, lens, tbl)


# speedup vs baseline: 4.7136x; 1.0011x over previous
import functools

import jax
import jax.numpy as jnp
from jax import lax
from jax.experimental import pallas as pl
from jax.experimental.pallas import tpu as pltpu
from jax.experimental.pallas import tpu_sc as plsc

_BATCH = 4096
_HIST = 200
_DIM = 32
_V1 = 1000001
_VP = 1000064
_CHUNK = 4
_ROWS = _CHUNK * _HIST
_K = 896
_C0_MAX = _VP - _K


def _make_transpose(nc, ns):
    mesh = plsc.VectorSubcoreMesh(core_axis_name="c", subcore_axis_name="s")
    nw = nc * ns
    n_chunks = -(-_VP // _K)
    n_slots = -(-n_chunks // nw)
    n_slots += n_slots % 2
    n_pairs = n_slots // 2

    @functools.partial(
        pl.kernel,
        mesh=mesh,
        compiler_params=pltpu.CompilerParams(
            disable_bounds_checks=True, needs_layout_passes=False),
        out_type=jax.ShapeDtypeStruct((_VP * _DIM,), jnp.float32),
        scratch_types=[
            pltpu.VMEM((_DIM, _K), jnp.float32),
            pltpu.VMEM((_DIM, _K), jnp.float32),
            pltpu.VMEM((_K * _DIM,), jnp.float32),
            pltpu.VMEM((_K * _DIM,), jnp.float32),
            pltpu.SemaphoreType.DMA,
            pltpu.SemaphoreType.DMA,
            pltpu.SemaphoreType.DMA,
            pltpu.SemaphoreType.DMA,
        ],
    )
    def k(tt_hbm, flat_hbm, buf0, buf1, out0, out1, si0, si1, so0, so1):
        wid = lax.axis_index("s") * nc + lax.axis_index("c")
        lane32 = lax.iota(jnp.int32, 16) * _DIM
        bufs = (buf0, buf1)
        outs = (out0, out1)
        sis = (si0, si1)
        sos = (so0, so1)

        def c_of(ci):
            return jnp.minimum((wid * n_slots + ci) * _K, _C0_MAX)

        def start_in(ci, b):
            pltpu.async_copy(tt_hbm.at[:, pl.ds(c_of(ci), _K)], bufs[b], sis[b])

        def wait_in(ci, b):
            pltpu.make_async_copy(
                tt_hbm.at[:, pl.ds(c_of(ci), _K)], bufs[b], sis[b]).wait()

        def start_out(ci, b):
            pltpu.async_copy(
                outs[b], flat_hbm.at[pl.ds(c_of(ci) * _DIM, _K * _DIM)], sos[b])

        def wait_out(ci, b):
            pltpu.make_async_copy(
                outs[b], flat_hbm.at[pl.ds(c_of(ci) * _DIM, _K * _DIM)],
                sos[b]).wait()

        def pair(g, carry):
            for b in range(2):
                ci = 2 * g + b

                start_in(ci, b)
                wait_in(ci, b)

                @pl.when(ci >= 2)
                def _():
                    wait_out(ci - 2, b)

                @functools.partial(
                    plsc.parallel_loop, 0, (_K // 16) * _DIM, unroll=16)
                def _(i):
                    d = i & (_DIM - 1)
                    gg = i >> 5
                    v = bufs[b][d, pl.ds(gg * 16, 16)]
                    plsc.store_scatter(
                        outs[b], [gg * (16 * _DIM) + lane32 + d], v)

                start_out(ci, b)
            return carry

        lax.fori_loop(0, n_pairs, pair, 0)
        wait_out(n_slots - 2, 0)
        wait_out(n_slots - 1, 1)

    return k


def _make_gather(nc, ns, bpw):
    mesh = plsc.VectorSubcoreMesh(core_axis_name="c", subcore_axis_name="s")
    n_chunks = bpw // _CHUNK

    @functools.partial(
        pl.kernel,
        mesh=mesh,
        compiler_params=pltpu.CompilerParams(use_tc_tiling_on_sc=False),
        out_type=jax.ShapeDtypeStruct((_BATCH, _DIM), jnp.float32),
        scratch_types=[
            pltpu.VMEM((bpw * _HIST,), jnp.int32),
            pltpu.VMEM((bpw, 16), jnp.float32),
            pltpu.VMEM((_ROWS, _DIM), jnp.float32),
            pltpu.VMEM((_ROWS, _DIM), jnp.float32),
            pltpu.VMEM((bpw, _DIM), jnp.float32),
            pltpu.SemaphoreType.DMA,
            pltpu.SemaphoreType.DMA,
        ],
    )
    def k(idx_hbm, lens_hbm, table_hbm, out_hbm,
          idx_v, lens_v, buf0, buf1, out_v, sem0, sem1):
        wid = lax.axis_index("s") * nc + lax.axis_index("c")
        base = wid * bpw
        pltpu.sync_copy(idx_hbm.at[wid], idx_v)
        pltpu.sync_copy(lens_hbm.at[pl.ds(base, bpw)], lens_v)

        def gather(cc, buf, sem):
            return pltpu.async_copy(
                table_hbm.at[idx_v.at[pl.ds(cc * _ROWS, _ROWS)]], buf, sem)

        gather(0, buf0, sem0)

        def super_body(g, carry):
            for b in range(2):
                cc = 2 * g + b
                bufc, semc = (buf0, sem0) if b == 0 else (buf1, sem1)
                bufn, semn = (buf1, sem1) if b == 0 else (buf0, sem0)

                @pl.when(cc + 1 < n_chunks)
                def _():
                    gather(cc + 1, bufn, semn)

                pltpu.make_async_copy(
                    table_hbm.at[idx_v.at[pl.ds(cc * _ROWS, _ROWS)]],
                    bufc, semc).wait()

                for jj in range(_CHUNK):
                    rbase = jj * _HIST

                    def red(l, acc):
                        accs = list(acc)
                        r0 = rbase + l * 8
                        for t in range(8):
                            p = t % 4
                            accs[2 * p] = accs[2 * p] + bufc[r0 + t, pl.ds(0, 16)]
                            accs[2 * p + 1] = (
                                accs[2 * p + 1] + bufc[r0 + t, pl.ds(16, 16)])
                        return tuple(accs)

                    zero = jnp.zeros((16,), jnp.float32)
                    accs = lax.fori_loop(0, _HIST // 8, red, (zero,) * 8)
                    a0 = (accs[0] + accs[2]) + (accs[4] + accs[6])
                    a1 = (accs[1] + accs[3]) + (accs[5] + accs[7])
                    j = cc * _CHUNK + jj
                    lenv = lens_v[j, pl.ds(0, 16)]
                    out_v[j, pl.ds(0, 16)] = a0 / lenv
                    out_v[j, pl.ds(16, 16)] = a1 / lenv
            return carry

        lax.fori_loop(0, n_chunks // 2, super_body, 0)
        pltpu.sync_copy(out_v, out_hbm.at[pl.ds(base, bpw)])

    return k


def kernel(inputs, input_lens, table):
    info = plsc.get_sparse_core_info()
    nc, ns = info.num_cores, info.num_subcores
    nw = nc * ns
    bpw = _BATCH // nw
    idx = inputs.reshape(nw, bpw * _HIST)
    lens = jnp.broadcast_to(input_lens.reshape(_BATCH, 1), (_BATCH, 16))
    tt = jnp.transpose(table)
    flat = _make_transpose(nc, ns)(tt)
    tbl = flat.reshape(_VP, _DIM)
    return _make_gather(nc, ns, bpw)(idx, lens, tbl)


# speedup vs baseline: 4.7171x; 1.0007x over previous
import functools

import jax
import jax.numpy as jnp
from jax import lax
from jax.experimental import pallas as pl
from jax.experimental.pallas import tpu as pltpu
from jax.experimental.pallas import tpu_sc as plsc

_BATCH = 4096
_HIST = 200
_DIM = 32
_V1 = 1000001
_VP = 1000064
_CHUNK = 4
_ROWS = _CHUNK * _HIST
_K = 896
_C0_MAX = _VP - _K


def _make_transpose(nc, ns):
    mesh = plsc.VectorSubcoreMesh(core_axis_name="c", subcore_axis_name="s")
    nw = nc * ns
    n_chunks = -(-_VP // _K)
    n_slots = -(-n_chunks // nw)
    n_slots += n_slots % 2
    n_pairs = n_slots // 2

    @functools.partial(
        pl.kernel,
        mesh=mesh,
        compiler_params=pltpu.CompilerParams(
            disable_bounds_checks=True, needs_layout_passes=False),
        out_type=jax.ShapeDtypeStruct((_VP * _DIM,), jnp.float32),
        scratch_types=[
            pltpu.VMEM((_DIM, _K), jnp.float32),
            pltpu.VMEM((_DIM, _K), jnp.float32),
            pltpu.VMEM((_K * _DIM,), jnp.float32),
            pltpu.VMEM((_K * _DIM,), jnp.float32),
            pltpu.SemaphoreType.DMA,
            pltpu.SemaphoreType.DMA,
            pltpu.SemaphoreType.DMA,
            pltpu.SemaphoreType.DMA,
        ],
    )
    def k(tt_hbm, flat_hbm, buf0, buf1, out0, out1, si0, si1, so0, so1):
        wid = lax.axis_index("s") * nc + lax.axis_index("c")
        lane32 = lax.iota(jnp.int32, 16) * _DIM
        bufs = (buf0, buf1)
        outs = (out0, out1)
        sis = (si0, si1)
        sos = (so0, so1)

        def c_of(ci):
            return jnp.minimum((wid * n_slots + ci) * _K, _C0_MAX)

        def start_in(ci, b):
            pltpu.async_copy(tt_hbm.at[:, pl.ds(c_of(ci), _K)], bufs[b], sis[b])

        def wait_in(ci, b):
            pltpu.make_async_copy(
                tt_hbm.at[:, pl.ds(c_of(ci), _K)], bufs[b], sis[b]).wait()

        def start_out(ci, b):
            pltpu.async_copy(
                outs[b], flat_hbm.at[pl.ds(c_of(ci) * _DIM, _K * _DIM)], sos[b])

        def wait_out(ci, b):
            pltpu.make_async_copy(
                outs[b], flat_hbm.at[pl.ds(c_of(ci) * _DIM, _K * _DIM)],
                sos[b]).wait()

        def pair(g, carry):
            for b in range(2):
                ci = 2 * g + b

                start_in(ci, b)
                wait_in(ci, b)

                @pl.when(ci >= 2)
                def _():
                    wait_out(ci - 2, b)

                @functools.partial(
                    plsc.parallel_loop, 0, (_K // 16) * _DIM, unroll=8)
                def _(i):
                    d = i & (_DIM - 1)
                    gg = i >> 5
                    v = bufs[b][d, pl.ds(gg * 16, 16)]
                    plsc.store_scatter(
                        outs[b], [gg * (16 * _DIM) + lane32 + d], v)

                start_out(ci, b)
            return carry

        lax.fori_loop(0, n_pairs, pair, 0)
        wait_out(n_slots - 2, 0)
        wait_out(n_slots - 1, 1)

    return k


def _make_gather(nc, ns, bpw):
    mesh = plsc.VectorSubcoreMesh(core_axis_name="c", subcore_axis_name="s")
    n_chunks = bpw // _CHUNK

    @functools.partial(
        pl.kernel,
        mesh=mesh,
        compiler_params=pltpu.CompilerParams(use_tc_tiling_on_sc=False),
        out_type=jax.ShapeDtypeStruct((_BATCH, _DIM), jnp.float32),
        scratch_types=[
            pltpu.VMEM((bpw * _HIST,), jnp.int32),
            pltpu.VMEM((bpw, 16), jnp.float32),
            pltpu.VMEM((_ROWS, _DIM), jnp.float32),
            pltpu.VMEM((_ROWS, _DIM), jnp.float32),
            pltpu.VMEM((bpw, _DIM), jnp.float32),
            pltpu.SemaphoreType.DMA,
            pltpu.SemaphoreType.DMA,
        ],
    )
    def k(idx_hbm, lens_hbm, table_hbm, out_hbm,
          idx_v, lens_v, buf0, buf1, out_v, sem0, sem1):
        wid = lax.axis_index("s") * nc + lax.axis_index("c")
        base = wid * bpw
        pltpu.sync_copy(idx_hbm.at[wid], idx_v)
        pltpu.sync_copy(lens_hbm.at[pl.ds(base, bpw)], lens_v)

        def gather(cc, buf, sem):
            return pltpu.async_copy(
                table_hbm.at[idx_v.at[pl.ds(cc * _ROWS, _ROWS)]], buf, sem)

        gather(0, buf0, sem0)

        def super_body(g, carry):
            for b in range(2):
                cc = 2 * g + b
                bufc, semc = (buf0, sem0) if b == 0 else (buf1, sem1)
                bufn, semn = (buf1, sem1) if b == 0 else (buf0, sem0)

                @pl.when(cc + 1 < n_chunks)
                def _():
                    gather(cc + 1, bufn, semn)

                pltpu.make_async_copy(
                    table_hbm.at[idx_v.at[pl.ds(cc * _ROWS, _ROWS)]],
                    bufc, semc).wait()

                for jj in range(_CHUNK):
                    rbase = jj * _HIST

                    def red(l, acc):
                        accs = list(acc)
                        r0 = rbase + l * 8
                        for t in range(8):
                            p = t % 4
                            accs[2 * p] = accs[2 * p] + bufc[r0 + t, pl.ds(0, 16)]
                            accs[2 * p + 1] = (
                                accs[2 * p + 1] + bufc[r0 + t, pl.ds(16, 16)])
                        return tuple(accs)

                    zero = jnp.zeros((16,), jnp.float32)
                    accs = lax.fori_loop(0, _HIST // 8, red, (zero,) * 8)
                    a0 = (accs[0] + accs[2]) + (accs[4] + accs[6])
                    a1 = (accs[1] + accs[3]) + (accs[5] + accs[7])
                    j = cc * _CHUNK + jj
                    lenv = lens_v[j, pl.ds(0, 16)]
                    out_v[j, pl.ds(0, 16)] = a0 / lenv
                    out_v[j, pl.ds(16, 16)] = a1 / lenv
            return carry

        lax.fori_loop(0, n_chunks // 2, super_body, 0)
        pltpu.sync_copy(out_v, out_hbm.at[pl.ds(base, bpw)])

    return k


def kernel(inputs, input_lens, table):
    info = plsc.get_sparse_core_info()
    nc, ns = info.num_cores, info.num_subcores
    nw = nc * ns
    bpw = _BATCH // nw
    idx = inputs.reshape(nw, bpw * _HIST)
    lens = jnp.broadcast_to(input_lens.reshape(_BATCH, 1), (_BATCH, 16))
    tt = jnp.transpose(table)
    flat = _make_transpose(nc, ns)(tt)
    tbl = flat.reshape(_VP, _DIM)
    return _make_gather(nc, ns, bpw)(idx, lens, tbl)
